# baseline (speedup 1.0000x reference)
import functools

import jax
import jax.numpy as jnp
from jax import lax
from jax.experimental import pallas as pl
from jax.experimental.pallas import tpu as pltpu
from jax.experimental.pallas import tpu_sc as plsc

D = 64
EPS = 1e-10
NW = 32
CHUNK = 128
FIRE = 7


def _sc_gather(item_table, rating_table, user_table, item_idx, rating_idx,
               uid_idx, nbr_idx):
    item_chunks = item_idx.shape[0] // (NW * CHUNK)
    nbr_chunks = nbr_idx.shape[0] // (NW * CHUNK)
    uid_per = uid_idx.shape[0] // NW

    mesh = plsc.VectorSubcoreMesh(core_axis_name="c", subcore_axis_name="s")

    @functools.partial(
        pl.kernel,
        out_type=[
            jax.ShapeDtypeStruct((item_chunks * NW, CHUNK, D), item_table.dtype),
            jax.ShapeDtypeStruct((item_chunks * NW, CHUNK, D), item_table.dtype),
            jax.ShapeDtypeStruct((uid_idx.shape[0], D), item_table.dtype),
            jax.ShapeDtypeStruct((nbr_chunks * NW, CHUNK, D), item_table.dtype),
        ],
        mesh=mesh,
        scratch_types=[
            pltpu.VMEM((item_chunks * CHUNK,), jnp.int32),
            pltpu.VMEM((FIRE, CHUNK, D), item_table.dtype),
            pltpu.VMEM((uid_per,), jnp.int32),
            pltpu.VMEM((uid_per, D), item_table.dtype),
            pltpu.SemaphoreType.DMA,
        ],
        compiler_params=pltpu.CompilerParams(use_tc_tiling_on_sc=False),
    )
    def gather_k(item_t, rating_t, user_t, item_i, rating_i, uid_i, nbr_i,
                 item_o, rating_o, uid_o, nbr_o,
                 idx_v, rows_v, uidx_v, urows_v, sem):
        wid = lax.axis_index("s") * 2 + lax.axis_index("c")

        def job(tab, idx_h, out_h, n_chunks):
            base = wid * n_chunks
            per_tile = n_chunks * CHUNK
            pltpu.sync_copy(idx_h.at[pl.ds(wid * per_tile, per_tile)],
                            idx_v.at[pl.ds(0, per_tile)])
            n_groups = n_chunks // FIRE
            rem = n_chunks - n_groups * FIRE

            def fire_drain(g, k):
                cps = [
                    pltpu.make_async_copy(
                        tab.at[idx_v.at[pl.ds((g * FIRE + j) * CHUNK, CHUNK)]],
                        rows_v.at[j], sem)
                    for j in range(k)
                ]
                for c in cps:
                    c.start()
                for c in cps:
                    c.wait()
                pltpu.sync_copy(rows_v.at[pl.ds(0, k)],
                                out_h.at[pl.ds(base + g * FIRE, k)])

            def body(g, carry):
                fire_drain(g, FIRE)
                return carry

            lax.fori_loop(0, n_groups, body, 0)
            if rem:
                fire_drain(n_groups, rem)

        job(item_t, item_i, item_o, item_chunks)
        job(rating_t, rating_i, rating_o, item_chunks)
        job(user_t, nbr_i, nbr_o, nbr_chunks)
        pltpu.sync_copy(uid_i.at[pl.ds(wid * uid_per, uid_per)], uidx_v)
        pltpu.async_copy(user_t.at[uidx_v], urows_v, sem).wait()
        pltpu.sync_copy(urows_v, uid_o.at[pl.ds(wid * uid_per, uid_per)])

    return gather_k(item_table, rating_table, user_table, item_idx,
                    rating_idx, uid_idx, nbr_idx)


def _seg_mat(per, rows, cols, dtype):
    c = lax.broadcasted_iota(jnp.int32, (rows, cols), 1)
    r = lax.broadcasted_iota(jnp.int32, (rows, cols), 0)
    return (c // per == r).astype(dtype)


def _rep_mat(per, rows, cols, dtype):
    c = lax.broadcasted_iota(jnp.int32, (rows, cols), 1)
    r = lax.broadcasted_iota(jnp.int32, (rows, cols), 0)
    return (r // per == c).astype(dtype)


def _tc_compute(BU, B, Li, Nn, H,
                item_g, rating_g, pu_g, nbr_g, mk1_i, mkE_i, mkO_i, mku_i,
                gvW1t, gvb1, gvW2t, gvb2, uiW1t, uib1, uiw2, uib2,
                aiWt, aib, uuW1t, uub1, uuw2, uub2, anWt, anb,
                mW1t, mb1, mW2t, mb2, mW3t, mb3):
    G = B // BU
    R1 = BU * Li
    R2 = BU * Nn * H
    RN = BU * Nn
    RT = R1 + 2 * R2
    EQ_OFF = (B * Li) // R2
    OQ_OFF = EQ_OFF + (B * Nn * H) // R2
    f32 = jnp.float32

    bf16 = jnp.bfloat16

    def dot(a, b):
        return lax.dot_general(a.astype(bf16), b.astype(bf16),
                               (((1,), (0,)), ((), ())),
                               preferred_element_type=f32)

    def body(qa, ra, eq, oq, eer, oer, pu, nbr, mk1i, mkEi, mkOi, mkui,
             gvW1t_r, gvb1_r, gvW2t_r, gvb2_r, uiW1t_r, uib1_r,
             uiw2_r, uib2_r, aiWt_r, aib_r, uuW1t_r, uub1_r, uuw2_r,
             uub2_r, anWt_r, anb_r, mW1t_r, mb1_r, mW2t_r, mb2_r,
             mW3t_r, mb3_r, out):
        relu = lambda x: jnp.maximum(x, 0.0)
        cat1 = lambda xs: jnp.concatenate(xs, axis=1)
        cat0 = lambda xs: jnp.concatenate(xs, axis=0)

        seg50 = _seg_mat(Li, BU, R1, bf16)
        rep50 = _rep_mat(Li, R1, BU, bf16)
        seg10 = _seg_mat(H, RN, R2, bf16)
        rep10 = _rep_mat(H, R2, RN, bf16)
        seg20 = _seg_mat(Nn, BU, RN, bf16)

        mk1 = (mk1i[...] > 0).astype(f32)
        mkE = (mkEi[...] > 0).astype(f32)
        mkO = (mkOi[...] > 0).astype(f32)
        mku = (mkui[...] > 0).astype(f32)
        pu_ = pu[...]
        nbr_ = nbr[...]

        X = cat0([cat1([qa[...], ra[...]]), cat1([eq[...], oq[...]]),
                  cat1([eer[...], oer[...]])])
        xh = relu(dot(X, gvW1t_r[...]) + gvb1_r[...])
        xall = dot(xh, gvW2t_r[...]) + gvb2_r[...]

        P = cat0([mk1 * dot(rep50, pu_),
                  mkE * dot(rep10, nbr_), mkO * dot(rep10, nbr_)])
        t = relu(dot(cat1([xall, P]), uiW1t_r[...]) + uib1_r[...])
        logit = jnp.sum(t * uiw2_r[...], axis=1, keepdims=True) + uib2_r[...]
        M = cat0([mk1, mkE, mkO])
        a = jnp.exp(logit) * M
        ax = a * xall

        den1 = dot(seg50, a[:R1]) + EPS
        wsum = dot(seg50, ax[:R1])
        den_s = dot(seg10, a[R1:R1 + R2] + a[R1 + R2:]) + EPS
        num = dot(seg10, ax[R1:R1 + R2] + ax[R1 + R2:])

        agg = cat0([wsum / den1, num / den_s])
        hagg = relu(dot(agg, aiWt_r[...]) + aib_r[...])
        h_iI = hagg[:BU]
        h_oI = hagg[BU:]

        bt = relu(dot(cat1([h_oI, nbr_]), uuW1t_r[...]) + uub1_r[...])
        bl = jnp.sum(bt * uuw2_r[...], axis=1, keepdims=True) + uub2_r[...]
        be = jnp.exp(bl) * mku
        den_b = dot(seg20, be) + EPS
        s2 = dot(seg20, be * h_oI) / den_b
        h_iS = relu(dot(s2, anWt_r[...]) + anb_r[...])

        h = relu(dot(cat1([h_iI, h_iS]), mW1t_r[...]) + mb1_r[...])
        h = relu(dot(h, mW2t_r[...]) + mb2_r[...])
        h = relu(dot(h, mW3t_r[...]) + mb3_r[...])
        out[...] = h

    def fixed(shape):
        return pl.BlockSpec(shape, lambda i: (0,) * len(shape))

    in_specs = [
        pl.BlockSpec((R1, D), lambda i: (i, 0)),
        pl.BlockSpec((R1, D), lambda i: (i, 0)),
        pl.BlockSpec((R2, D), lambda i: (EQ_OFF + i, 0)),
        pl.BlockSpec((R2, D), lambda i: (OQ_OFF + i, 0)),
        pl.BlockSpec((R2, D), lambda i: (EQ_OFF + i, 0)),
        pl.BlockSpec((R2, D), lambda i: (OQ_OFF + i, 0)),
        pl.BlockSpec((BU, D), lambda i: (i, 0)),
        pl.BlockSpec((RN, D), lambda i: (i, 0)),
        pl.BlockSpec((R1, 1), lambda i: (i, 0)),
        pl.BlockSpec((R2, 1), lambda i: (i, 0)),
        pl.BlockSpec((R2, 1), lambda i: (i, 0)),
        pl.BlockSpec((RN, 1), lambda i: (i, 0)),
    ] + [fixed(w.shape) for w in (
        gvW1t, gvb1, gvW2t, gvb2, uiW1t, uib1, uiw2, uib2,
        aiWt, aib, uuW1t, uub1, uuw2, uub2, anWt, anb,
        mW1t, mb1, mW2t, mb2, mW3t, mb3)]

    return pl.pallas_call(
        body,
        grid=(G,),
        in_specs=in_specs,
        out_specs=pl.BlockSpec((BU, D), lambda i: (i, 0)),
        out_shape=jax.ShapeDtypeStruct((B, D), jnp.float32),
    )(item_g, rating_g, item_g, item_g, rating_g, rating_g, pu_g, nbr_g,
      mk1_i, mkE_i, mkO_i, mku_i,
      gvW1t, gvb1, gvW2t, gvb2, uiW1t, uib1, uiw2, uib2,
      aiWt, aib, uuW1t, uub1, uuw2, uub2, anWt, anb,
      mW1t, mb1, mW2t, mb2, mW3t, mb3)


def kernel(uids, u_item_pad, u_user_pad, u_user_item_pad, user_table,
           item_table, rating_table, gv_W1, gv_b1, gv_W2, gv_b2, ui_W1,
           ui_b1, ui_W2, ui_b2, ai_W, ai_b, uu_W1, uu_b1, uu_W2, uu_b2,
           an_W, an_b, m_W1, m_b1, m_W2, m_b2, m_W3, m_b3):
    B, Li, _ = u_item_pad.shape
    _, Nn, Mi, _ = u_user_item_pad.shape
    H = Mi // 2
    i32 = jnp.int32

    uip = u_item_pad.astype(i32)
    uuip = u_user_item_pad.astype(i32)
    item_idx = jnp.concatenate([
        uip[:, :, 0].reshape(-1),
        uuip[:, :, 0::2, 0].reshape(-1), uuip[:, :, 1::2, 0].reshape(-1)])
    rating_idx = jnp.concatenate([
        uip[:, :, 1].reshape(-1),
        uuip[:, :, 0::2, 1].reshape(-1), uuip[:, :, 1::2, 1].reshape(-1)])
    n_raw = item_idx.shape[0]
    n_pad = -n_raw % (NW * CHUNK)
    if n_pad:
        pad = jnp.zeros((n_pad,), i32)
        item_idx = jnp.concatenate([item_idx, pad])
        rating_idx = jnp.concatenate([rating_idx, pad])
    nbr_idx = u_user_pad.astype(i32).reshape(-1)
    uid_idx = uids.astype(i32)

    bf16 = jnp.bfloat16
    item_g3, rating_g3, pu_g, nbr_g3 = _sc_gather(
        item_table.astype(bf16), rating_table.astype(bf16),
        user_table.astype(bf16), item_idx, rating_idx, uid_idx, nbr_idx)
    item_g = item_g3.reshape(-1, D)
    rating_g = rating_g3.reshape(-1, D)
    nbr_g = nbr_g3.reshape(-1, D)

    mk1_i = uip[:, :, 0].reshape(-1, 1)
    mkE_i = uuip[:, :, :H, 0].reshape(-1, 1)
    mkO_i = uuip[:, :, H:, 0].reshape(-1, 1)
    mku_i = u_user_pad.astype(i32).reshape(-1, 1)

    row = lambda b: b.reshape(1, -1)
    return _tc_compute(
        8, B, Li, Nn, H,
        item_g, rating_g, pu_g, nbr_g, mk1_i, mkE_i, mkO_i, mku_i,
        gv_W1.T, row(gv_b1), gv_W2.T, row(gv_b2),
        ui_W1.T, row(ui_b1), ui_W2, row(ui_b2),
        ai_W.T, row(ai_b),
        uu_W1.T, row(uu_b1), uu_W2, row(uu_b2),
        an_W.T, row(an_b),
        m_W1.T, row(m_b1), m_W2.T, row(m_b2),
        m_W3.T, row(m_b3))


# speedup vs baseline: 1.1032x; 1.1032x over previous
import functools

import jax
import jax.numpy as jnp
from jax import lax
from jax.experimental import pallas as pl
from jax.experimental.pallas import tpu as pltpu
from jax.experimental.pallas import tpu_sc as plsc

D = 64
EPS = 1e-10
NW = 32
CHUNK = 128
FIRE = 7
BU1 = 16
BU2 = 128


def _sc_gather(comb_table, user_table, left_idx, right_idx, uid_idx, nbr_idx):
    x_chunks = left_idx.shape[0] // (NW * CHUNK)
    nbr_chunks = nbr_idx.shape[0] // (NW * CHUNK)
    uid_per = uid_idx.shape[0] // NW

    mesh = plsc.VectorSubcoreMesh(core_axis_name="c", subcore_axis_name="s")

    @functools.partial(
        pl.kernel,
        out_type=[
            jax.ShapeDtypeStruct((x_chunks * NW, CHUNK, D), comb_table.dtype),
            jax.ShapeDtypeStruct((x_chunks * NW, CHUNK, D), comb_table.dtype),
            jax.ShapeDtypeStruct((uid_idx.shape[0], D), comb_table.dtype),
            jax.ShapeDtypeStruct((nbr_chunks * NW, CHUNK, D), comb_table.dtype),
        ],
        mesh=mesh,
        scratch_types=[
            pltpu.VMEM((x_chunks * CHUNK,), jnp.int32),
            pltpu.VMEM((FIRE, CHUNK, D), comb_table.dtype),
            pltpu.VMEM((uid_per,), jnp.int32),
            pltpu.VMEM((uid_per, D), comb_table.dtype),
            pltpu.SemaphoreType.DMA,
        ],
        compiler_params=pltpu.CompilerParams(use_tc_tiling_on_sc=False),
    )
    def gather_k(comb_t, user_t, left_i, right_i, uid_i, nbr_i,
                 left_o, right_o, uid_o, nbr_o,
                 idx_v, rows_v, uidx_v, urows_v, sem):
        wid = lax.axis_index("s") * 2 + lax.axis_index("c")

        def job(tab, idx_h, out_h, n_chunks):
            base = wid * n_chunks
            per_tile = n_chunks * CHUNK
            pltpu.sync_copy(idx_h.at[pl.ds(wid * per_tile, per_tile)],
                            idx_v.at[pl.ds(0, per_tile)])
            n_groups = n_chunks // FIRE
            rem = n_chunks - n_groups * FIRE

            def fire_drain(g, k):
                cps = [
                    pltpu.make_async_copy(
                        tab.at[idx_v.at[pl.ds((g * FIRE + j) * CHUNK, CHUNK)]],
                        rows_v.at[j], sem)
                    for j in range(k)
                ]
                for c in cps:
                    c.start()
                for c in cps:
                    c.wait()
                pltpu.sync_copy(rows_v.at[pl.ds(0, k)],
                                out_h.at[pl.ds(base + g * FIRE, k)])

            def body(g, carry):
                fire_drain(g, FIRE)
                return carry

            lax.fori_loop(0, n_groups, body, 0)
            if rem:
                fire_drain(n_groups, rem)

        job(comb_t, left_i, left_o, x_chunks)
        job(comb_t, right_i, right_o, x_chunks)
        job(user_t, nbr_i, nbr_o, nbr_chunks)
        pltpu.sync_copy(uid_i.at[pl.ds(wid * uid_per, uid_per)], uidx_v)
        pltpu.async_copy(user_t.at[uidx_v], urows_v, sem).wait()
        pltpu.sync_copy(urows_v, uid_o.at[pl.ds(wid * uid_per, uid_per)])

    return gather_k(comb_table, user_table, left_idx, right_idx, uid_idx,
                    nbr_idx)


def _seg_mat(per, rows, cols, dtype):
    c = lax.broadcasted_iota(jnp.int32, (rows, cols), 1)
    r = lax.broadcasted_iota(jnp.int32, (rows, cols), 0)
    return (c // per == r).astype(dtype)


def _rep_mat2(per, rows, cols, dtype):
    cc = lax.broadcasted_iota(jnp.int32, (rows, cols), 1)
    rr = lax.broadcasted_iota(jnp.int32, (rows, cols), 0)
    return ((rr % (per * cols)) // per == cc).astype(dtype)


def _tc_stage1(B, Li, Nn, H,
               xl, xr, mk, pu_g, nbr_g, seg_li, rep_li, seg_h, rep_h2,
               gvW1a, gvW1b, gvb1, gvW2t, gvb2, uiWf, uibf, uiWp,
               uiw2c, uib2):
    G = B // BU1
    R1 = BU1 * Li
    R2 = BU1 * Nn * H
    RN = BU1 * Nn
    RT = R1 + 2 * R2
    f32 = jnp.float32
    bf16 = jnp.bfloat16

    def dot(a, b):
        return lax.dot_general(a.astype(bf16), b.astype(bf16),
                               (((1,), (0,)), ((), ())),
                               preferred_element_type=f32)

    def body(xl_r, xr_r, mk_r, pu_r, nbr_r,
             seg_li_r, rep_li_r, seg_h_r, rep_h2_r,
             gvW1a_r, gvW1b_r, gvb1_r, gvW2t_r, gvb2_r, uiWf_r, uibf_r,
             uiWp_r, uiw2c_r, uib2_r, o1, o2):
        relu = lambda x: jnp.maximum(x, 0.0)
        seg_li = seg_li_r[...]
        rep_li = rep_li_r[...]
        seg_h = seg_h_r[...]
        rep_h2 = rep_h2_r[...]

        mkf = (mk_r[...] > 0).astype(f32)
        xh = relu(dot(xl_r[...], gvW1a_r[...]) +
                  dot(xr_r[...], gvW1b_r[...]) + gvb1_r[...])
        xall = dot(xh, gvW2t_r[...]) + gvb2_r[...]
        pre = dot(xh, uiWf_r[...]) + uibf_r[...]
        s_u = dot(pu_r[...], uiWp_r[...])
        s_n = dot(nbr_r[...], uiWp_r[...])
        u_b = dot(rep_li, s_u)
        n_b = dot(rep_h2, s_n)

        t1 = relu(pre[:R1] + mkf[:R1] * u_b)
        ts = relu(pre[R1:] + mkf[R1:] * n_b)
        a1 = jnp.exp(dot(t1, uiw2c_r[...]) + uib2_r[...]) * mkf[:R1]
        as_ = jnp.exp(dot(ts, uiw2c_r[...]) + uib2_r[...]) * mkf[R1:]
        ax1 = a1 * xall[:R1]
        axs = as_ * xall[R1:]

        den1 = dot(seg_li, a1) + EPS
        wsum = dot(seg_li, ax1)
        den_s = dot(seg_h, as_[:R2] + as_[R2:]) + EPS
        num = dot(seg_h, axs[:R2] + axs[R2:])
        o1[...] = wsum / den1
        o2[...] = num / den_s

    def fixed(shape):
        return pl.BlockSpec(shape, lambda i: (0,) * len(shape))

    in_specs = [
        pl.BlockSpec((RT, D), lambda i: (i, 0)),
        pl.BlockSpec((RT, D), lambda i: (i, 0)),
        pl.BlockSpec((RT, 1), lambda i: (i, 0)),
        pl.BlockSpec((BU1, D), lambda i: (i, 0)),
        pl.BlockSpec((RN, D), lambda i: (i, 0)),
    ] + [fixed(w.shape) for w in (
        seg_li, rep_li, seg_h, rep_h2,
        gvW1a, gvW1b, gvb1, gvW2t, gvb2, uiWf, uibf, uiWp, uiw2c, uib2)]

    return pl.pallas_call(
        body,
        grid=(G,),
        in_specs=in_specs,
        out_specs=[pl.BlockSpec((BU1, D), lambda i: (i, 0)),
                   pl.BlockSpec((RN, D), lambda i: (i, 0))],
        out_shape=[jax.ShapeDtypeStruct((B, D), jnp.float32),
                   jax.ShapeDtypeStruct((B * Nn, D), jnp.float32)],
    )(xl, xr, mk, pu_g, nbr_g, seg_li, rep_li, seg_h, rep_h2,
      gvW1a, gvW1b, gvb1, gvW2t, gvb2, uiWf, uibf, uiWp, uiw2c, uib2)


def _tc_stage2(B, Nn,
               o1, o2, nbr_g, mku, seg_nn,
               aiWt, aib, uuW1a, uuW1b, uub1, uuw2c, uub2, anWt, anb,
               mW1a, mW1b, mb1, mW2t, mb2, mW3t, mb3):
    G = B // BU2
    RN = BU2 * Nn
    f32 = jnp.float32
    bf16 = jnp.bfloat16

    def dot(a, b):
        return lax.dot_general(a.astype(bf16), b.astype(bf16),
                               (((1,), (0,)), ((), ())),
                               preferred_element_type=f32)

    def body(o1_r, o2_r, nbr_r, mku_r, seg_nn_r,
             aiWt_r, aib_r, uuW1a_r, uuW1b_r, uub1_r, uuw2c_r, uub2_r,
             anWt_r, anb_r, mW1a_r, mW1b_r, mb1_r, mW2t_r, mb2_r,
             mW3t_r, mb3_r, out):
        relu = lambda x: jnp.maximum(x, 0.0)
        seg_nn = seg_nn_r[...]

        h_iI = relu(dot(o1_r[...], aiWt_r[...]) + aib_r[...])
        h_oI = relu(dot(o2_r[...], aiWt_r[...]) + aib_r[...])
        bt = relu(dot(h_oI, uuW1a_r[...]) +
                  dot(nbr_r[...], uuW1b_r[...]) + uub1_r[...])
        bl = dot(bt, uuw2c_r[...]) + uub2_r[...]
        be = jnp.exp(bl) * (mku_r[...] > 0).astype(f32)
        den_b = dot(seg_nn, be) + EPS
        s2 = dot(seg_nn, be * h_oI) / den_b
        h_iS = relu(dot(s2, anWt_r[...]) + anb_r[...])

        h = relu(dot(h_iI, mW1a_r[...]) + dot(h_iS, mW1b_r[...]) + mb1_r[...])
        h = relu(dot(h, mW2t_r[...]) + mb2_r[...])
        out[...] = relu(dot(h, mW3t_r[...]) + mb3_r[...])

    def fixed(shape):
        return pl.BlockSpec(shape, lambda i: (0,) * len(shape))

    in_specs = [
        pl.BlockSpec((BU2, D), lambda i: (i, 0)),
        pl.BlockSpec((RN, D), lambda i: (i, 0)),
        pl.BlockSpec((RN, D), lambda i: (i, 0)),
        pl.BlockSpec((RN, 1), lambda i: (i, 0)),
    ] + [fixed(w.shape) for w in (
        seg_nn, aiWt, aib, uuW1a, uuW1b, uub1, uuw2c, uub2, anWt, anb,
        mW1a, mW1b, mb1, mW2t, mb2, mW3t, mb3)]

    return pl.pallas_call(
        body,
        grid=(G,),
        in_specs=in_specs,
        out_specs=pl.BlockSpec((BU2, D), lambda i: (i, 0)),
        out_shape=jax.ShapeDtypeStruct((B, D), jnp.float32),
    )(o1, o2, nbr_g, mku, seg_nn,
      aiWt, aib, uuW1a, uuW1b, uub1, uuw2c, uub2, anWt, anb,
      mW1a, mW1b, mb1, mW2t, mb2, mW3t, mb3)


def kernel(uids, u_item_pad, u_user_pad, u_user_item_pad, user_table,
           item_table, rating_table, gv_W1, gv_b1, gv_W2, gv_b2, ui_W1,
           ui_b1, ui_W2, ui_b2, ai_W, ai_b, uu_W1, uu_b1, uu_W2, uu_b2,
           an_W, an_b, m_W1, m_b1, m_W2, m_b2, m_W3, m_b3):
    B, Li, _ = u_item_pad.shape
    _, Nn, Mi, _ = u_user_item_pad.shape
    H = Mi // 2
    V = item_table.shape[0]
    i32 = jnp.int32
    G1 = B // BU1

    uip = u_item_pad.astype(i32)
    uuip = u_user_item_pad.astype(i32)
    left = jnp.concatenate([
        uip[:, :, 0].reshape(G1, -1),
        uuip[:, :, 0::2, 0].reshape(G1, -1),
        uuip[:, :, 0::2, 1].reshape(G1, -1) + V], axis=1).reshape(-1)
    right = jnp.concatenate([
        uip[:, :, 1].reshape(G1, -1) + V,
        uuip[:, :, 1::2, 0].reshape(G1, -1),
        uuip[:, :, 1::2, 1].reshape(G1, -1) + V], axis=1).reshape(-1)
    mk = jnp.concatenate([
        uip[:, :, 0].reshape(G1, -1),
        uuip[:, :, :H, 0].reshape(G1, -1),
        uuip[:, :, H:, 0].reshape(G1, -1)], axis=1).reshape(-1, 1)
    n_raw = left.shape[0]
    n_pad = -n_raw % (NW * CHUNK)
    if n_pad:
        pad = jnp.zeros((n_pad,), i32)
        left = jnp.concatenate([left, pad])
        right = jnp.concatenate([right, pad])
    nbr_idx = u_user_pad.astype(i32).reshape(-1)
    uid_idx = uids.astype(i32)
    mku = u_user_pad.astype(i32).reshape(-1, 1)

    bf16 = jnp.bfloat16
    comb = jnp.concatenate([item_table, rating_table]).astype(bf16)
    xl3, xr3, pu_g, nbr_g3 = _sc_gather(
        comb, user_table.astype(bf16), left, right, uid_idx, nbr_idx)
    xl = xl3.reshape(-1, D)
    xr = xr3.reshape(-1, D)
    nbr_g = nbr_g3.reshape(-1, D)

    R1 = BU1 * Li
    R2 = BU1 * Nn * H
    RN1 = BU1 * Nn
    seg_li = _seg_mat(Li, BU1, R1, bf16)
    rep_li = _rep_mat2(Li, R1, BU1, bf16)
    seg_h = _seg_mat(H, RN1, R2, bf16)
    rep_h2 = _rep_mat2(H, 2 * R2, RN1, bf16)
    seg_nn = _seg_mat(Nn, BU2, BU2 * Nn, bf16)

    row = lambda b: b.reshape(1, -1)
    uiW1x = ui_W1.T[:D]
    uiWf = gv_W2.T @ uiW1x
    uibf = row(gv_b2 @ uiW1x + ui_b1)
    o1, o2 = _tc_stage1(
        B, Li, Nn, H, xl, xr, mk, pu_g, nbr_g,
        seg_li, rep_li, seg_h, rep_h2,
        gv_W1.T[:D], gv_W1.T[D:], row(gv_b1), gv_W2.T, row(gv_b2),
        uiWf, uibf, ui_W1.T[D:], ui_W2.T, row(ui_b2))
    return _tc_stage2(
        B, Nn, o1, o2, nbr_g, mku, seg_nn,
        ai_W.T, row(ai_b), uu_W1.T[:D], uu_W1.T[D:], row(uu_b1),
        uu_W2.T, row(uu_b2), an_W.T, row(an_b),
        m_W1.T[:D], m_W1.T[D:], row(m_b1), m_W2.T, row(m_b2),
        m_W3.T, row(m_b3))


# speedup vs baseline: 1.1945x; 1.0828x over previous
import functools

import jax
import jax.numpy as jnp
from jax import lax
from jax.experimental import pallas as pl
from jax.experimental.pallas import tpu as pltpu
from jax.experimental.pallas import tpu_sc as plsc

D = 64
EPS = 1e-10
NW = 32
CHUNK = 128
FIRE = 7
BU1 = 16
BU2 = 128


def _sc_gather(comb_table, user_table, left_idx, right_idx, uid_idx, nbr_idx):
    x_chunks = left_idx.shape[0] // (NW * CHUNK)
    nbr_chunks = nbr_idx.shape[0] // (NW * CHUNK)
    uid_per = uid_idx.shape[0] // NW

    mesh = plsc.VectorSubcoreMesh(core_axis_name="c", subcore_axis_name="s")

    @functools.partial(
        pl.kernel,
        out_type=[
            jax.ShapeDtypeStruct((x_chunks * NW, CHUNK, D), comb_table.dtype),
            jax.ShapeDtypeStruct((x_chunks * NW, CHUNK, D), comb_table.dtype),
            jax.ShapeDtypeStruct((uid_idx.shape[0], D), comb_table.dtype),
            jax.ShapeDtypeStruct((nbr_chunks * NW, CHUNK, D), comb_table.dtype),
        ],
        mesh=mesh,
        scratch_types=[
            pltpu.VMEM((x_chunks * CHUNK,), jnp.int32),
            pltpu.VMEM((FIRE, CHUNK, D), comb_table.dtype),
            pltpu.VMEM((uid_per,), jnp.int32),
            pltpu.VMEM((uid_per, D), comb_table.dtype),
            pltpu.SemaphoreType.DMA,
        ],
        compiler_params=pltpu.CompilerParams(use_tc_tiling_on_sc=False),
    )
    def gather_k(comb_t, user_t, left_i, right_i, uid_i, nbr_i,
                 left_o, right_o, uid_o, nbr_o,
                 idx_v, rows_v, uidx_v, urows_v, sem):
        wid = lax.axis_index("s") * 2 + lax.axis_index("c")

        def job(tab, idx_h, out_h, n_chunks):
            base = wid * n_chunks
            per_tile = n_chunks * CHUNK
            pltpu.sync_copy(idx_h.at[pl.ds(wid * per_tile, per_tile)],
                            idx_v.at[pl.ds(0, per_tile)])
            n_groups = n_chunks // FIRE
            rem = n_chunks - n_groups * FIRE

            def fire_drain(g, k):
                cps = [
                    pltpu.make_async_copy(
                        tab.at[idx_v.at[pl.ds((g * FIRE + j) * CHUNK, CHUNK)]],
                        rows_v.at[j], sem)
                    for j in range(k)
                ]
                for c in cps:
                    c.start()
                for c in cps:
                    c.wait()
                pltpu.sync_copy(rows_v.at[pl.ds(0, k)],
                                out_h.at[pl.ds(base + g * FIRE, k)])

            def body(g, carry):
                fire_drain(g, FIRE)
                return carry

            lax.fori_loop(0, n_groups, body, 0)
            if rem:
                fire_drain(n_groups, rem)

        job(comb_t, left_i, left_o, x_chunks)
        job(comb_t, right_i, right_o, x_chunks)
        job(user_t, nbr_i, nbr_o, nbr_chunks)
        pltpu.sync_copy(uid_i.at[pl.ds(wid * uid_per, uid_per)], uidx_v)
        pltpu.async_copy(user_t.at[uidx_v], urows_v, sem).wait()
        pltpu.sync_copy(urows_v, uid_o.at[pl.ds(wid * uid_per, uid_per)])

    return gather_k(comb_table, user_table, left_idx, right_idx, uid_idx,
                    nbr_idx)


def _seg_mat(per, rows, cols, dtype):
    c = lax.broadcasted_iota(jnp.int32, (rows, cols), 1)
    r = lax.broadcasted_iota(jnp.int32, (rows, cols), 0)
    return (c // per == r).astype(dtype)


def _rep_mat2(per, rows, cols, dtype):
    cc = lax.broadcasted_iota(jnp.int32, (rows, cols), 1)
    rr = lax.broadcasted_iota(jnp.int32, (rows, cols), 0)
    return ((rr % (per * cols)) // per == cc).astype(dtype)


def _tc_stage1(B, Li, Nn, H,
               xl, xr, mk1, mks, pu_g, nbr_g, seg_li, rep_li, seg_h, rep_h2,
               gvW1a, gvW1b, gvb1, gvW2t, gvb2, uiWf, uibf, uiWp,
               uiw2c, uib2):
    G = B // BU1
    R1 = BU1 * Li
    R2 = BU1 * Nn * H
    RN = BU1 * Nn
    RT = R1 + 2 * R2
    f32 = jnp.float32
    bf16 = jnp.bfloat16

    def dot(a, b):
        return lax.dot_general(a.astype(bf16), b.astype(bf16),
                               (((1,), (0,)), ((), ())),
                               preferred_element_type=f32)

    def body(xl_r, xr_r, mk1_r, mks_r, pu_r, nbr_r,
             seg_li_r, rep_li_r, seg_h_r, rep_h2_r,
             gvW1a_r, gvW1b_r, gvb1_r, gvW2t_r, gvb2_r, uiWf_r, uibf_r,
             uiWp_r, uiw2c_r, uib2_r, o1, o2):
        relu = lambda x: jnp.maximum(x, 0.0)
        seg_li = seg_li_r[...]
        rep_li = rep_li_r[...]
        seg_h = seg_h_r[...]
        rep_h2 = rep_h2_r[...]

        mk1c = mk1_r[...]
        mksc = mks_r[...]
        xh = relu(dot(xl_r[...], gvW1a_r[...]) +
                  dot(xr_r[...], gvW1b_r[...]) + gvb1_r[...])
        xall = dot(xh, gvW2t_r[...]) + gvb2_r[...]
        pre = dot(xh, uiWf_r[...]) + uibf_r[...]
        s_u = dot(pu_r[...], uiWp_r[...])
        s_n = dot(nbr_r[...], uiWp_r[...])
        u_b = dot(rep_li, s_u)
        n_b = dot(rep_h2, s_n)

        t1 = relu(pre[:R1] + mk1c * u_b)
        ts = relu(pre[R1:] + mksc * n_b)
        a1 = jnp.exp(dot(t1, uiw2c_r[...]) + uib2_r[...]) * mk1c
        as_ = jnp.exp(dot(ts, uiw2c_r[...]) + uib2_r[...]) * mksc
        ax1 = a1 * xall[:R1]
        axs = as_ * xall[R1:]

        den1 = dot(seg_li, a1) + EPS
        wsum = dot(seg_li, ax1)
        den_s = dot(seg_h, as_[:R2] + as_[R2:]) + EPS
        num = dot(seg_h, axs[:R2] + axs[R2:])
        o1[...] = wsum / den1
        o2[...] = num / den_s

    def fixed(shape):
        return pl.BlockSpec(shape, lambda i: (0,) * len(shape))

    in_specs = [
        pl.BlockSpec((RT, D), lambda i: (i, 0)),
        pl.BlockSpec((RT, D), lambda i: (i, 0)),
        pl.BlockSpec((R1, 1), lambda i: (i, 0)),
        pl.BlockSpec((2 * R2, 1), lambda i: (i, 0)),
        pl.BlockSpec((BU1, D), lambda i: (i, 0)),
        pl.BlockSpec((RN, D), lambda i: (i, 0)),
    ] + [fixed(w.shape) for w in (
        seg_li, rep_li, seg_h, rep_h2,
        gvW1a, gvW1b, gvb1, gvW2t, gvb2, uiWf, uibf, uiWp, uiw2c, uib2)]

    return pl.pallas_call(
        body,
        grid=(G,),
        in_specs=in_specs,
        out_specs=[pl.BlockSpec((BU1, D), lambda i: (i, 0)),
                   pl.BlockSpec((RN, D), lambda i: (i, 0))],
        out_shape=[jax.ShapeDtypeStruct((B, D), jnp.float32),
                   jax.ShapeDtypeStruct((B * Nn, D), jnp.float32)],
    )(xl, xr, mk1, mks, pu_g, nbr_g, seg_li, rep_li, seg_h, rep_h2,
      gvW1a, gvW1b, gvb1, gvW2t, gvb2, uiWf, uibf, uiWp, uiw2c, uib2)


def _tc_stage2(B, Nn,
               o1, o2, nbr_g, mku, seg_nn,
               aiWt, aib, uuW1a, uuW1b, uub1, uuw2c, uub2, anWt, anb,
               mW1a, mW1b, mb1, mW2t, mb2, mW3t, mb3):
    G = B // BU2
    RN = BU2 * Nn
    f32 = jnp.float32
    bf16 = jnp.bfloat16

    def dot(a, b):
        return lax.dot_general(a.astype(bf16), b.astype(bf16),
                               (((1,), (0,)), ((), ())),
                               preferred_element_type=f32)

    def body(o1_r, o2_r, nbr_r, mku_r, seg_nn_r,
             aiWt_r, aib_r, uuW1a_r, uuW1b_r, uub1_r, uuw2c_r, uub2_r,
             anWt_r, anb_r, mW1a_r, mW1b_r, mb1_r, mW2t_r, mb2_r,
             mW3t_r, mb3_r, out):
        relu = lambda x: jnp.maximum(x, 0.0)
        seg_nn = seg_nn_r[...]

        h_iI = relu(dot(o1_r[...], aiWt_r[...]) + aib_r[...])
        h_oI = relu(dot(o2_r[...], aiWt_r[...]) + aib_r[...])
        bt = relu(dot(h_oI, uuW1a_r[...]) +
                  dot(nbr_r[...], uuW1b_r[...]) + uub1_r[...])
        bl = dot(bt, uuw2c_r[...]) + uub2_r[...]
        be = jnp.exp(bl) * mku_r[...]
        den_b = dot(seg_nn, be) + EPS
        s2 = dot(seg_nn, be * h_oI) / den_b
        h_iS = relu(dot(s2, anWt_r[...]) + anb_r[...])

        h = relu(dot(h_iI, mW1a_r[...]) + dot(h_iS, mW1b_r[...]) + mb1_r[...])
        h = relu(dot(h, mW2t_r[...]) + mb2_r[...])
        out[...] = relu(dot(h, mW3t_r[...]) + mb3_r[...])

    def fixed(shape):
        return pl.BlockSpec(shape, lambda i: (0,) * len(shape))

    in_specs = [
        pl.BlockSpec((BU2, D), lambda i: (i, 0)),
        pl.BlockSpec((RN, D), lambda i: (i, 0)),
        pl.BlockSpec((RN, D), lambda i: (i, 0)),
        pl.BlockSpec((RN, 1), lambda i: (i, 0)),
    ] + [fixed(w.shape) for w in (
        seg_nn, aiWt, aib, uuW1a, uuW1b, uub1, uuw2c, uub2, anWt, anb,
        mW1a, mW1b, mb1, mW2t, mb2, mW3t, mb3)]

    return pl.pallas_call(
        body,
        grid=(G,),
        in_specs=in_specs,
        out_specs=pl.BlockSpec((BU2, D), lambda i: (i, 0)),
        out_shape=jax.ShapeDtypeStruct((B, D), jnp.float32),
    )(o1, o2, nbr_g, mku, seg_nn,
      aiWt, aib, uuW1a, uuW1b, uub1, uuw2c, uub2, anWt, anb,
      mW1a, mW1b, mb1, mW2t, mb2, mW3t, mb3)


def kernel(uids, u_item_pad, u_user_pad, u_user_item_pad, user_table,
           item_table, rating_table, gv_W1, gv_b1, gv_W2, gv_b2, ui_W1,
           ui_b1, ui_W2, ui_b2, ai_W, ai_b, uu_W1, uu_b1, uu_W2, uu_b2,
           an_W, an_b, m_W1, m_b1, m_W2, m_b2, m_W3, m_b3):
    B, Li, _ = u_item_pad.shape
    _, Nn, Mi, _ = u_user_item_pad.shape
    H = Mi // 2
    V = item_table.shape[0]
    i32 = jnp.int32
    G1 = B // BU1

    uip = u_item_pad.astype(i32)
    uuip = u_user_item_pad.astype(i32)
    left = jnp.concatenate([
        uip[:, :, 0].reshape(G1, -1),
        uuip[:, :, 0::2, 0].reshape(G1, -1),
        uuip[:, :, 0::2, 1].reshape(G1, -1) + V], axis=1).reshape(-1)
    right = jnp.concatenate([
        uip[:, :, 1].reshape(G1, -1) + V,
        uuip[:, :, 1::2, 0].reshape(G1, -1),
        uuip[:, :, 1::2, 1].reshape(G1, -1) + V], axis=1).reshape(-1)
    f32 = jnp.float32
    mk1 = (uip[:, :, 0] > 0).astype(f32).reshape(B * Li, 1)
    ms3 = (uuip[:, :, :, 0] > 0).astype(f32).reshape(G1, BU1 * Nn, Mi)
    mks = jnp.concatenate([ms3[:, :, :H].reshape(G1, -1),
                           ms3[:, :, H:].reshape(G1, -1)],
                          axis=1).reshape(-1, 1)
    n_raw = left.shape[0]
    n_pad = -n_raw % (NW * CHUNK)
    if n_pad:
        pad = jnp.zeros((n_pad,), i32)
        left = jnp.concatenate([left, pad])
        right = jnp.concatenate([right, pad])
    nbr_idx = u_user_pad.astype(i32).reshape(-1)
    uid_idx = uids.astype(i32)
    mku = (u_user_pad > 0).astype(f32).reshape(B * Nn, 1)

    bf16 = jnp.bfloat16
    comb = jnp.concatenate([item_table, rating_table]).astype(bf16)
    xl3, xr3, pu_g, nbr_g3 = _sc_gather(
        comb, user_table.astype(bf16), left, right, uid_idx, nbr_idx)
    xl = xl3.reshape(-1, D)
    xr = xr3.reshape(-1, D)
    nbr_g = nbr_g3.reshape(-1, D)

    R1 = BU1 * Li
    R2 = BU1 * Nn * H
    RN1 = BU1 * Nn
    seg_li = _seg_mat(Li, BU1, R1, bf16)
    rep_li = _rep_mat2(Li, R1, BU1, bf16)
    seg_h = _seg_mat(H, RN1, R2, bf16)
    rep_h2 = _rep_mat2(H, 2 * R2, RN1, bf16)
    seg_nn = _seg_mat(Nn, BU2, BU2 * Nn, bf16)

    row = lambda b: b.reshape(1, -1)
    uiW1x = ui_W1.T[:D]
    uiWf = gv_W2.T @ uiW1x
    uibf = row(gv_b2 @ uiW1x + ui_b1)
    o1, o2 = _tc_stage1(
        B, Li, Nn, H, xl, xr, mk1, mks, pu_g, nbr_g,
        seg_li, rep_li, seg_h, rep_h2,
        gv_W1.T[:D], gv_W1.T[D:], row(gv_b1), gv_W2.T, row(gv_b2),
        uiWf, uibf, ui_W1.T[D:], ui_W2.T, row(ui_b2))
    return _tc_stage2(
        B, Nn, o1, o2, nbr_g, mku, seg_nn,
        ai_W.T, row(ai_b), uu_W1.T[:D], uu_W1.T[D:], row(uu_b1),
        uu_W2.T, row(uu_b2), an_W.T, row(an_b),
        m_W1.T[:D], m_W1.T[D:], row(m_b1), m_W2.T, row(m_b2),
        m_W3.T, row(m_b3))
